# baseline (speedup 1.0000x reference)
import functools

import jax
import jax.numpy as jnp
from jax import lax
from jax.experimental import pallas as pl
from jax.experimental.pallas import tpu as pltpu
from jax.experimental.pallas import tpu_sc as plsc

N = 10000
D = 128
E = 320000
NT = 16
RPT = 624
CH = 128
NCHUNK = 160
NPH = 10
CPP = NCHUNK // NPH
EP = NT * NCHUNK * CH
DUMMY = N
NACC = 10008
NPASS = 3
TS = 3344
TPT = 208
GH = 64

RB = 1024
GRID = 10
NTBL = RB * GRID
RRB = 1000

_f32 = jnp.float32


def _mm1_body(hd, hp, wdd, wpd, wpp, wdp, mpd_o, mdp_o, d1_o, p1_o):
    mpd_o[...] = jnp.dot(hp[...], wpd[...], preferred_element_type=_f32)
    mdp_o[...] = jnp.dot(hd[...], wdp[...], preferred_element_type=_f32)
    d1_o[...] = jnp.dot(hd[...], wdd[...], preferred_element_type=_f32)
    p1_o[...] = jnp.dot(hp[...], wpp[...], preferred_element_type=_f32)


def _mid_body(sd, sp, hd, hp, wdd, wpd, wpp, wdp, mpd_o, mdp_o, d2_o, p2_o):
    xd = jnp.maximum(sd[...], 0.0)
    xp = jnp.maximum(sp[...], 0.0)
    mpd_o[...] = jnp.dot(xp, wpd[...], preferred_element_type=_f32)
    mdp_o[...] = jnp.dot(xd, wdp[...], preferred_element_type=_f32)
    d2_o[...] = jnp.dot(xd, wdd[...], preferred_element_type=_f32) + hd[...]
    p2_o[...] = jnp.dot(xp, wpp[...], preferred_element_type=_f32) + hp[...]


def _relu_body(sd, sp, od, op):
    od[...] = jnp.maximum(sd[...], 0.0)
    op[...] = jnp.maximum(sp[...], 0.0)


_row_spec = pl.BlockSpec((RB, D), lambda i: (i, 0))
_rrow_spec = pl.BlockSpec((RRB, D), lambda i: (i, 0))
_w_spec = pl.BlockSpec((D, D), lambda i: (0, 0))
_nd = jax.ShapeDtypeStruct((N, D), _f32)
_td = jax.ShapeDtypeStruct((NTBL, D), _f32)

_mm1 = pl.pallas_call(
    _mm1_body,
    grid=(GRID,),
    in_specs=[_row_spec, _row_spec] + [_w_spec] * 4,
    out_specs=[_row_spec] * 4,
    out_shape=[_td] * 4,
)

_mid = pl.pallas_call(
    _mid_body,
    grid=(GRID,),
    in_specs=[_row_spec] * 4 + [_w_spec] * 4,
    out_specs=[_row_spec] * 4,
    out_shape=[_td] * 4,
)

_relu2 = pl.pallas_call(
    _relu_body,
    grid=(GRID,),
    in_specs=[_rrow_spec, _rrow_spec],
    out_specs=[_rrow_spec, _rrow_spec],
    out_shape=[_nd] * 2,
)


@functools.partial(
    pl.kernel,
    mesh=plsc.VectorSubcoreMesh(core_axis_name="c", subcore_axis_name="s"),
    out_type=[_nd, _nd],
    scratch_types=[
        pltpu.VMEM_SHARED((NACC, D), _f32),
        pltpu.VMEM_SHARED((TS, D), _f32),
        pltpu.VMEM((CPP, CH), jnp.int32),
        pltpu.VMEM((CPP, CH), jnp.int32),
        pltpu.VMEM((GH, D), _f32),
        pltpu.VMEM((GH, D), _f32),
        pltpu.SemaphoreType.DMA,
        pltpu.SemaphoreType.DMA,
        pltpu.SemaphoreType.DMA,
        pltpu.SemaphoreType.DMA,
    ],
)
def _sc_agg(mpd_hbm, mdp_hbm, didx_hbm, pidx_hbm, dinit_hbm, pinit_hbm,
            outd_hbm, outp_hbm, acc, tbl_sp, gidx_v, sidx_v,
            rows0, rows1, semg0, semg1, sems0, sems1):
    c = lax.axis_index("c")
    s = lax.axis_index("s")
    rbase = s * RPT

    def run_dir(tbl, gidx, sidx, init, out):
        pltpu.sync_copy(init.at[pl.ds(rbase, RPT)],
                        acc.at[pl.ds(rbase, RPT)])

        @pl.when(s == NT - 1)
        def _():
            pltpu.sync_copy(init.at[pl.ds(NT * RPT, N - NT * RPT)],
                            acc.at[pl.ds(NT * RPT, N - NT * RPT)])

        def gather(j, h, rows, semg):
            pltpu.async_copy(tbl_sp.at[gidx_v.at[j, pl.ds(h * GH, GH)]],
                             rows, semg)

        def scatter(j, h, rows, sems):
            pltpu.async_copy(rows, acc.at[sidx_v.at[j, pl.ds(h * GH, GH)]],
                             sems, add=True)

        def wait_g(rows, semg):
            pltpu.make_async_copy(tbl_sp.at[gidx_v.at[0, pl.ds(0, GH)]],
                                  rows, semg).wait()

        def wait_s(rows, sems):
            pltpu.make_async_copy(rows, acc.at[sidx_v.at[0, pl.ds(0, GH)]],
                                  sems).wait()

        for ps in range(NPASS):
            lo = ps * TS
            plsc.subcore_barrier()
            pltpu.sync_copy(tbl.at[pl.ds(lo + s * TPT, TPT)],
                            tbl_sp.at[pl.ds(s * TPT, TPT)])

            @pl.when(s == NT - 1)
            def _():
                pltpu.sync_copy(tbl.at[pl.ds(lo + NT * TPT, TS - NT * TPT)],
                                tbl_sp.at[pl.ds(NT * TPT, TS - NT * TPT)])

            plsc.subcore_barrier()

            def phase(p, carry):
                pltpu.sync_copy(gidx.at[s].at[pl.ds(p * CPP, CPP)], gidx_v)
                pltpu.sync_copy(sidx.at[s].at[pl.ds(p * CPP, CPP)], sidx_v)

                def clamp(r, carry2):
                    for v in range(8):
                        g = gidx_v[r, pl.ds(v * 16, 16)]
                        inb = (g >= lo) & (g < lo + TS)
                        gidx_v[r, pl.ds(v * 16, 16)] = jnp.where(inb, g - lo, 0)
                        sv = sidx_v[r, pl.ds(v * 16, 16)]
                        sidx_v[r, pl.ds(v * 16, 16)] = jnp.where(inb, sv, DUMMY)
                    return carry2

                carry = lax.fori_loop(0, CPP, clamp, carry)

                gather(0, 0, rows0, semg0)

                def body(k, carry2):
                    wait_g(rows0, semg0)
                    scatter(k, 0, rows0, sems0)

                    @pl.when(k > 0)
                    def _():
                        wait_s(rows1, sems1)

                    gather(k, 1, rows1, semg1)
                    wait_g(rows1, semg1)
                    scatter(k, 1, rows1, sems1)
                    wait_s(rows0, sems0)

                    @pl.when(k < CPP - 1)
                    def _():
                        gather(k + 1, 0, rows0, semg0)

                    return carry2

                carry = lax.fori_loop(0, CPP, body, carry)
                wait_s(rows1, sems1)
                return carry

            lax.fori_loop(0, NPH, phase, 0)

        plsc.subcore_barrier()
        pltpu.sync_copy(acc.at[pl.ds(rbase, RPT)],
                        out.at[pl.ds(rbase, RPT)])

        @pl.when(s == NT - 1)
        def _():
            pltpu.sync_copy(acc.at[pl.ds(NT * RPT, N - NT * RPT)],
                            out.at[pl.ds(NT * RPT, N - NT * RPT)])

    @pl.when(c == 0)
    def _():
        run_dir(mpd_hbm, pidx_hbm, didx_hbm, dinit_hbm, outd_hbm)

    @pl.when(c == 1)
    def _():
        run_dir(mdp_hbm, didx_hbm, pidx_hbm, pinit_hbm, outp_hbm)


def kernel(h_drug, h_prot, edge_index,
           W1_dd, W1_pd, W1_pp, W1_dp,
           W2_dd, W2_pd, W2_pp, W2_dp):
    pad = jnp.full((EP - E,), DUMMY, jnp.int32)
    didx = jnp.concatenate([edge_index[0], pad]).reshape(NT, NCHUNK, CH)
    pidx = jnp.concatenate([edge_index[1], pad]).reshape(NT, NCHUNK, CH)

    m1pd, m1dp, d1, p1 = _mm1(h_drug, h_prot, W1_dd, W1_pd, W1_pp, W1_dp)
    s1d, s1p = _sc_agg(m1pd, m1dp, didx, pidx, d1, p1)
    m2pd, m2dp, d2, p2 = _mid(s1d, s1p, h_drug, h_prot,
                              W2_dd, W2_pd, W2_pp, W2_dp)
    s2d, s2p = _sc_agg(m2pd, m2dp, didx, pidx, d2, p2)
    return tuple(_relu2(s2d, s2p))


# speedup vs baseline: 1.5108x; 1.5108x over previous
import functools

import jax
import jax.numpy as jnp
from jax import lax
from jax.experimental import pallas as pl
from jax.experimental.pallas import tpu as pltpu
from jax.experimental.pallas import tpu_sc as plsc

N = 10000
D = 128
E = 320000
NT = 16
RPT = 624
CH = 128
NCHUNK = 160
NPH = 4
CPP = NCHUNK // NPH
NIT = CPP // 2
EP = NT * NCHUNK * CH
DUMMY = N
NACC = 10016

RB = 1024
GRID = 10
NTBL = RB * GRID
RRB = 1000

_f32 = jnp.float32


def _mm1_body(hd, hp, wdd, wpd, wpp, wdp, mpd_o, mdp_o, d1_o, p1_o):
    mpd_o[...] = jnp.dot(hp[...], wpd[...], preferred_element_type=_f32)
    mdp_o[...] = jnp.dot(hd[...], wdp[...], preferred_element_type=_f32)
    d1_o[...] = jnp.dot(hd[...], wdd[...], preferred_element_type=_f32)
    p1_o[...] = jnp.dot(hp[...], wpp[...], preferred_element_type=_f32)


def _mid_body(sd, sp, hd, hp, wdd, wpd, wpp, wdp, mpd_o, mdp_o, d2_o, p2_o):
    xd = jnp.maximum(sd[...], 0.0)
    xp = jnp.maximum(sp[...], 0.0)
    mpd_o[...] = jnp.dot(xp, wpd[...], preferred_element_type=_f32)
    mdp_o[...] = jnp.dot(xd, wdp[...], preferred_element_type=_f32)
    d2_o[...] = jnp.dot(xd, wdd[...], preferred_element_type=_f32) + hd[...]
    p2_o[...] = jnp.dot(xp, wpp[...], preferred_element_type=_f32) + hp[...]


_row_spec = pl.BlockSpec((RB, D), lambda i: (i, 0))
_rrow_spec = pl.BlockSpec((RRB, D), lambda i: (i, 0))
_w_spec = pl.BlockSpec((D, D), lambda i: (0, 0))
_nd = jax.ShapeDtypeStruct((N, D), _f32)
_td = jax.ShapeDtypeStruct((NTBL, D), _f32)

_mm1 = pl.pallas_call(
    _mm1_body,
    grid=(GRID,),
    in_specs=[_row_spec, _row_spec] + [_w_spec] * 4,
    out_specs=[_row_spec] * 4,
    out_shape=[_td] * 4,
)

_mid = pl.pallas_call(
    _mid_body,
    grid=(GRID,),
    in_specs=[_row_spec] * 4 + [_w_spec] * 4,
    out_specs=[_row_spec] * 4,
    out_shape=[_td] * 4,
)


@functools.partial(
    pl.kernel,
    mesh=plsc.VectorSubcoreMesh(core_axis_name="c", subcore_axis_name="s"),
    out_type=[_nd, _nd],
    scratch_types=[
        pltpu.VMEM_SHARED((NACC, D), _f32),
        pltpu.VMEM((CPP, CH), jnp.int32),
        pltpu.VMEM((CPP, CH), jnp.int32),
        pltpu.VMEM((CH, D), _f32),
        pltpu.VMEM((CH, D), _f32),
        pltpu.SemaphoreType.DMA,
        pltpu.SemaphoreType.DMA,
        pltpu.SemaphoreType.DMA,
        pltpu.SemaphoreType.DMA,
    ],
)
def _sc_agg(mpd_hbm, mdp_hbm, didx_hbm, pidx_hbm, dinit_hbm, pinit_hbm,
            outd_hbm, outp_hbm, acc, gidx_v, sidx_v, rows0, rows1,
            semg0, semg1, sems0, sems1):
    c = lax.axis_index("c")
    s = lax.axis_index("s")
    rbase = s * RPT

    def run_dir(tbl, gidx, sidx, init, out):
        pltpu.sync_copy(init.at[pl.ds(rbase, RPT)],
                        acc.at[pl.ds(rbase, RPT)])

        @pl.when(s == NT - 1)
        def _():
            pltpu.sync_copy(init.at[pl.ds(NT * RPT, N - NT * RPT)],
                            acc.at[pl.ds(NT * RPT, N - NT * RPT)])

        plsc.subcore_barrier()

        def gather(j, rows, semg):
            pltpu.async_copy(tbl.at[gidx_v.at[j]], rows, semg)

        def scatter(j, rows, sems):
            pltpu.async_copy(rows, acc.at[sidx_v.at[j]], sems, add=True)

        def wait_g(rows, semg):
            pltpu.make_async_copy(tbl.at[gidx_v.at[0]], rows, semg).wait()

        def wait_s(rows, sems):
            pltpu.make_async_copy(rows, acc.at[sidx_v.at[0]], sems).wait()

        def phase(p, carry):
            pltpu.sync_copy(gidx.at[s].at[pl.ds(p * CPP, CPP)], gidx_v)
            pltpu.sync_copy(sidx.at[s].at[pl.ds(p * CPP, CPP)], sidx_v)
            gather(0, rows0, semg0)

            def body(k, carry2):
                j0 = 2 * k
                wait_g(rows0, semg0)
                scatter(j0, rows0, sems0)

                @pl.when(k > 0)
                def _():
                    wait_s(rows1, sems1)

                gather(j0 + 1, rows1, semg1)
                wait_g(rows1, semg1)
                scatter(j0 + 1, rows1, sems1)
                wait_s(rows0, sems0)

                @pl.when(k < NIT - 1)
                def _():
                    gather(j0 + 2, rows0, semg0)

                return carry2

            carry = lax.fori_loop(0, NIT, body, carry)
            wait_s(rows1, sems1)
            return carry

        lax.fori_loop(0, NPH, phase, 0)
        plsc.subcore_barrier()

        def wb_block(base, nrows):
            pltpu.sync_copy(acc.at[pl.ds(base, nrows)],
                            rows0.at[pl.ds(0, nrows)])

            def relu_row(r, carry3):
                for v in range(8):
                    x = rows0[r, pl.ds(v * 16, 16)]
                    rows0[r, pl.ds(v * 16, 16)] = jnp.maximum(x, 0.0)
                return carry3

            lax.fori_loop(0, nrows, relu_row, 0)
            pltpu.sync_copy(rows0.at[pl.ds(0, nrows)],
                            out.at[pl.ds(base, nrows)])

        def wb(b, carry2):
            wb_block(rbase + b * 104, 104)
            return carry2

        lax.fori_loop(0, RPT // 104, wb, 0)

        @pl.when(s == NT - 1)
        def _():
            wb_block(NT * RPT, N - NT * RPT)

    @pl.when(c == 0)
    def _():
        run_dir(mpd_hbm, pidx_hbm, didx_hbm, dinit_hbm, outd_hbm)

    @pl.when(c == 1)
    def _():
        run_dir(mdp_hbm, didx_hbm, pidx_hbm, pinit_hbm, outp_hbm)


def kernel(h_drug, h_prot, edge_index,
           W1_dd, W1_pd, W1_pp, W1_dp,
           W2_dd, W2_pd, W2_pp, W2_dp):
    pad = jnp.full((EP - E,), DUMMY, jnp.int32)
    didx = jnp.concatenate([edge_index[0], pad]).reshape(NT, NCHUNK, CH)
    pidx = jnp.concatenate([edge_index[1], pad]).reshape(NT, NCHUNK, CH)

    m1pd, m1dp, d1, p1 = _mm1(h_drug, h_prot, W1_dd, W1_pd, W1_pp, W1_dp)
    s1d, s1p = _sc_agg(m1pd, m1dp, didx, pidx, d1, p1)
    m2pd, m2dp, d2, p2 = _mid(s1d, s1p, h_drug, h_prot,
                              W2_dd, W2_pd, W2_pp, W2_dp)
    s2d, s2p = _sc_agg(m2pd, m2dp, didx, pidx, d2, p2)
    return (s2d, s2p)
